# baseline (speedup 1.0000x reference)
import functools

import jax
import jax.numpy as jnp
from jax import lax
from jax.experimental import pallas as pl
from jax.experimental.pallas import tpu as pltpu
from jax.experimental.pallas import tpu_sc as plsc

B = 16384
D_CONTEXT = 2048
D_TASK = 1024
D_ATTN = 1024
N_MODULES = 64
K_ACTIVE = 8

TILE = 512

NC, NS, L = 2, 16, 16
NW = NC * NS

N_CHUNKS = 1
B_CHUNK = B // N_CHUNKS
ROWS_PER_W = B_CHUNK // NW

_DOT_KW = dict(preferred_element_type=jnp.float32)


def _gater_body(c0_ref, c1_ref, e0_ref, e1_ref, Wc_ref, We_ref, ba_ref,
                Wa_ref, phi_ref):
    for t, (c_ref, e_ref) in enumerate(((c0_ref, e0_ref),
                                        (c1_ref, e1_ref))):
        pre = (jax.lax.dot_general(c_ref[...], Wc_ref[...],
                                   (((1,), (1,)), ((), ())), **_DOT_KW)
               + jax.lax.dot_general(e_ref[...], We_ref[...],
                                     (((1,), (1,)), ((), ())), **_DOT_KW)
               + ba_ref[...])
        h = jnp.tanh(pre)
        A = jax.lax.dot_general(h, Wa_ref[...], (((1,), (1,)), ((), ())),
                                **_DOT_KW)
        m = jnp.max(A, axis=1, keepdims=True)
        ex = jnp.exp(A - m)
        phi_ref[t * TILE:(t + 1) * TILE, :] = (
            ex / jnp.sum(ex, axis=1, keepdims=True))


_sc_mesh = plsc.VectorSubcoreMesh(core_axis_name="c", subcore_axis_name="s",
                                  num_cores=NC, num_subcores=NS)


def _router_tail_body(phi_hbm, phik_hbm, idx_hbm, phi_v, phik_v, idx_v):
    wid = lax.axis_index("s") * NC + lax.axis_index("c")
    base = wid * ROWS_PER_W * N_MODULES
    ibase = wid * ROWS_PER_W * K_ACTIVE
    pltpu.sync_copy(phi_hbm.at[pl.ds(base, ROWS_PER_W * N_MODULES)], phi_v)

    lanes = lax.iota(jnp.int32, L)
    low8 = lanes < K_ACTIVE
    zero16 = jnp.zeros((L,), jnp.float32)

    def row(r, carry):
        off = r * N_MODULES

        sk, sv = [], []
        for j in range(4):
            key = phi_v[pl.ds(off + 16 * j, 16)]
            kj, vj = plsc.sort_key_val(key, jnp.int32(16 * j) + lanes,
                                       descending=(j % 2 == 0))
            sk.append(kj)
            sv.append(vj)
        k01, v01 = plsc.sort_key_val(
            jnp.where(low8, sk[0], sk[1]),
            jnp.where(low8, sv[0], sv[1]), descending=True)
        k23, v23 = plsc.sort_key_val(
            jnp.where(low8, sk[2], sk[3]),
            jnp.where(low8, sv[2], sv[3]), descending=False)
        fk, fv = plsc.sort_key_val(
            jnp.where(low8, k01, k23),
            jnp.where(low8, v01, v23), descending=True)

        vals8 = jnp.where(low8, fk, 0.0)
        scaled = vals8 / (jnp.sum(vals8) + 1e-8)
        for j in range(4):
            phik_v[pl.ds(off + 16 * j, 16)] = zero16
        plsc.store_scatter(phik_v, [off + fv], scaled, mask=low8)
        plsc.store_scatter(idx_v, [r * K_ACTIVE + lanes], fv, mask=low8)
        return carry

    @plsc.parallel_loop(0, ROWS_PER_W, unroll=2)
    def _rows(r):
        row(r, 0)

    pltpu.sync_copy(phik_v, phik_hbm.at[pl.ds(base, ROWS_PER_W * N_MODULES)])
    pltpu.sync_copy(idx_v, idx_hbm.at[pl.ds(ibase, ROWS_PER_W * K_ACTIVE)])


_router_tail = functools.partial(
    pl.kernel,
    out_type=(jax.ShapeDtypeStruct((B_CHUNK * N_MODULES,), jnp.float32),
              jax.ShapeDtypeStruct((B_CHUNK * K_ACTIVE,), jnp.int32)),
    mesh=_sc_mesh,
    compiler_params=pltpu.CompilerParams(needs_layout_passes=False),
    scratch_types=[pltpu.VMEM((ROWS_PER_W * N_MODULES,), jnp.float32),
                   pltpu.VMEM((ROWS_PER_W * N_MODULES,), jnp.float32),
                   pltpu.VMEM((ROWS_PER_W * K_ACTIVE,), jnp.int32)],
)(_router_tail_body)


def _gater_chunk(c, e, Wc, We, ba2d, Wa):
    grid = (B_CHUNK // (2 * TILE),)
    return pl.pallas_call(
        _gater_body,
        grid=grid,
        in_specs=[
            pl.BlockSpec((TILE, D_CONTEXT), lambda i: (2 * i, 0)),
            pl.BlockSpec((TILE, D_CONTEXT), lambda i: (2 * i + 1, 0)),
            pl.BlockSpec((TILE, D_TASK), lambda i: (2 * i, 0)),
            pl.BlockSpec((TILE, D_TASK), lambda i: (2 * i + 1, 0)),
            pl.BlockSpec((D_ATTN, D_CONTEXT), lambda i: (0, 0)),
            pl.BlockSpec((D_ATTN, D_TASK), lambda i: (0, 0)),
            pl.BlockSpec((1, D_ATTN), lambda i: (0, 0)),
            pl.BlockSpec((N_MODULES, D_ATTN), lambda i: (0, 0)),
        ],
        out_specs=pl.BlockSpec((2 * TILE, N_MODULES), lambda i: (i, 0)),
        out_shape=jax.ShapeDtypeStruct((B_CHUNK, N_MODULES), jnp.float32),
        compiler_params=pltpu.CompilerParams(
            dimension_semantics=("parallel",)),
    )(c, c, e, e, Wc, We, ba2d, Wa)


@jax.jit
def kernel(c, e, Wc, We, ba, Wa):
    ba2d = ba.reshape(1, D_ATTN)
    phis, phiks, idxs = [], [], []
    for i in range(N_CHUNKS):
        sl = slice(i * B_CHUNK, (i + 1) * B_CHUNK)
        phi = _gater_chunk(c[sl], e[sl], Wc, We, ba2d, Wa)
        phik_flat, idx_flat = _router_tail(phi.reshape(B_CHUNK * N_MODULES))
        phis.append(phi)
        phiks.append(phik_flat.reshape(B_CHUNK, N_MODULES))
        idxs.append(idx_flat.reshape(B_CHUNK, K_ACTIVE))
    return (jnp.concatenate(phis, axis=0), jnp.concatenate(phiks, axis=0),
            jnp.concatenate(idxs, axis=0))


# speedup vs baseline: 1.0121x; 1.0121x over previous
import functools

import jax
import jax.numpy as jnp
from jax import lax
from jax.experimental import pallas as pl
from jax.experimental.pallas import tpu as pltpu
from jax.experimental.pallas import tpu_sc as plsc

B = 16384
D_CONTEXT = 2048
D_TASK = 1024
D_ATTN = 1024
N_MODULES = 64
K_ACTIVE = 8

TILE = 1024

NC, NS, L = 2, 16, 16
NW = NC * NS

N_CHUNKS = 1
B_CHUNK = B // N_CHUNKS
ROWS_PER_W = B_CHUNK // NW

_DOT_KW = dict(preferred_element_type=jnp.float32)


def _gater_body(c_ref, e_ref, Wc_ref, We_ref, ba_ref, Wa_ref, phi_ref):
    pre = (jax.lax.dot_general(c_ref[...], Wc_ref[...],
                               (((1,), (1,)), ((), ())), **_DOT_KW)
           + jax.lax.dot_general(e_ref[...], We_ref[...],
                                 (((1,), (1,)), ((), ())), **_DOT_KW)
           + ba_ref[...])
    h = jnp.tanh(pre)
    A = jax.lax.dot_general(h, Wa_ref[...], (((1,), (1,)), ((), ())),
                            **_DOT_KW)
    m = jnp.max(A, axis=1, keepdims=True)
    ex = jnp.exp(A - m)
    phi_ref[...] = ex / jnp.sum(ex, axis=1, keepdims=True)


_sc_mesh = plsc.VectorSubcoreMesh(core_axis_name="c", subcore_axis_name="s",
                                  num_cores=NC, num_subcores=NS)


def _router_tail_body(phi_hbm, phik_hbm, idx_hbm, phi_v, phik_v, idx_v):
    wid = lax.axis_index("s") * NC + lax.axis_index("c")
    base = wid * ROWS_PER_W * N_MODULES
    ibase = wid * ROWS_PER_W * K_ACTIVE
    pltpu.sync_copy(phi_hbm.at[pl.ds(base, ROWS_PER_W * N_MODULES)], phi_v)

    lanes = lax.iota(jnp.int32, L)
    low8 = lanes < K_ACTIVE
    zero16 = jnp.zeros((L,), jnp.float32)

    def row(r, carry):
        off = r * N_MODULES

        sk, sv = [], []
        for j in range(4):
            key = phi_v[pl.ds(off + 16 * j, 16)]
            kj, vj = plsc.sort_key_val(key, jnp.int32(16 * j) + lanes,
                                       descending=(j % 2 == 0))
            sk.append(kj)
            sv.append(vj)
        k01, v01 = plsc.sort_key_val(
            jnp.where(low8, sk[0], sk[1]),
            jnp.where(low8, sv[0], sv[1]), descending=True)
        k23, v23 = plsc.sort_key_val(
            jnp.where(low8, sk[2], sk[3]),
            jnp.where(low8, sv[2], sv[3]), descending=False)
        fk, fv = plsc.sort_key_val(
            jnp.where(low8, k01, k23),
            jnp.where(low8, v01, v23), descending=True)

        vals8 = jnp.where(low8, fk, 0.0)
        scaled = vals8 / (jnp.sum(vals8) + 1e-8)
        for j in range(4):
            phik_v[pl.ds(off + 16 * j, 16)] = zero16
        plsc.store_scatter(phik_v, [off + fv], scaled, mask=low8)
        plsc.store_scatter(idx_v, [r * K_ACTIVE + lanes], fv, mask=low8)
        return carry

    @plsc.parallel_loop(0, ROWS_PER_W, unroll=2)
    def _rows(r):
        row(r, 0)

    pltpu.sync_copy(phik_v, phik_hbm.at[pl.ds(base, ROWS_PER_W * N_MODULES)])
    pltpu.sync_copy(idx_v, idx_hbm.at[pl.ds(ibase, ROWS_PER_W * K_ACTIVE)])


_router_tail = functools.partial(
    pl.kernel,
    out_type=(jax.ShapeDtypeStruct((B_CHUNK * N_MODULES,), jnp.float32),
              jax.ShapeDtypeStruct((B_CHUNK * K_ACTIVE,), jnp.int32)),
    mesh=_sc_mesh,
    compiler_params=pltpu.CompilerParams(needs_layout_passes=False),
    scratch_types=[pltpu.VMEM((ROWS_PER_W * N_MODULES,), jnp.float32),
                   pltpu.VMEM((ROWS_PER_W * N_MODULES,), jnp.float32),
                   pltpu.VMEM((ROWS_PER_W * K_ACTIVE,), jnp.int32)],
)(_router_tail_body)


def _gater_chunk(c, e, Wc, We, ba2d, Wa):
    grid = (B_CHUNK // TILE,)
    return pl.pallas_call(
        _gater_body,
        grid=grid,
        in_specs=[
            pl.BlockSpec((TILE, D_CONTEXT), lambda i: (i, 0)),
            pl.BlockSpec((TILE, D_TASK), lambda i: (i, 0)),
            pl.BlockSpec((D_ATTN, D_CONTEXT), lambda i: (0, 0)),
            pl.BlockSpec((D_ATTN, D_TASK), lambda i: (0, 0)),
            pl.BlockSpec((1, D_ATTN), lambda i: (0, 0)),
            pl.BlockSpec((N_MODULES, D_ATTN), lambda i: (0, 0)),
        ],
        out_specs=pl.BlockSpec((TILE, N_MODULES), lambda i: (i, 0)),
        out_shape=jax.ShapeDtypeStruct((B_CHUNK, N_MODULES), jnp.float32),
        compiler_params=pltpu.CompilerParams(
            dimension_semantics=("parallel",)),
    )(c, e, Wc, We, ba2d, Wa)


@jax.jit
def kernel(c, e, Wc, We, ba, Wa):
    ba2d = ba.reshape(1, D_ATTN)
    phis, phiks, idxs = [], [], []
    for i in range(N_CHUNKS):
        sl = slice(i * B_CHUNK, (i + 1) * B_CHUNK)
        phi = _gater_chunk(c[sl], e[sl], Wc, We, ba2d, Wa)
        phik_flat, idx_flat = _router_tail(phi.reshape(B_CHUNK * N_MODULES))
        phis.append(phi)
        phiks.append(phik_flat.reshape(B_CHUNK, N_MODULES))
        idxs.append(idx_flat.reshape(B_CHUNK, K_ACTIVE))
    return (jnp.concatenate(phis, axis=0), jnp.concatenate(phiks, axis=0),
            jnp.concatenate(idxs, axis=0))
